# Initial kernel scaffold; baseline (speedup 1.0000x reference)
import jax
import jax.numpy as jnp
from jax.experimental import pallas as pl


def kernel(x, params):
    raise NotImplementedError("write your pallas kernel here")


# speedup vs baseline: 7.2612x; 7.2612x over previous
import functools

import numpy as np
import jax
import jax.numpy as jnp
from jax.experimental import pallas as pl
from jax.experimental.pallas import tpu as pltpu

_B, _CIN, _L = 16, 2, 4096
_F32 = jnp.float32
_BF16 = jnp.bfloat16


def _fwd_dft(L, m):
    n = np.arange(L, dtype=np.float64)[:, None]
    k = np.arange(m, dtype=np.float64)[None, :]
    ang = -2.0 * np.pi * n * k / L
    return np.cos(ang) / L, np.sin(ang) / L


def _inv_dft(m, N, nyq=None):
    k = np.arange(m, dtype=np.float64)[:, None]
    j = np.arange(N, dtype=np.float64)[None, :]
    ang = 2.0 * np.pi * k * j / N
    w = np.full((m, 1), 2.0)
    w[0, 0] = 1.0
    if nyq is not None:
        w[nyq, 0] = 1.0
    return w * np.cos(ang), -w * np.sin(ang)


def _resample_mat(Lin, Lout):
    m = min(Lin // 2 + 1, Lout // 2 + 1)
    cf, sf = _fwd_dft(Lin, m)
    nyq = m - 1 if (m - 1) == Lout // 2 else None
    ar, ai = _inv_dft(m, Lout, nyq=nyq)
    r = cf.astype(np.float32) @ ar.astype(np.float32)
    r += sf.astype(np.float32) @ ai.astype(np.float32)
    return r


def _cs_mat(L, m):
    cf, sf = _fwd_dft(L, m)
    return np.concatenate([cf, sf], axis=1).astype(np.float32)


def _aa_mat(m, N):
    ar, ai = _inv_dft(m, N)
    return np.concatenate([ar, ai], axis=0).astype(np.float32)


_R_DOWN = _resample_mat(4096, 2048)
_R_UP = _resample_mat(2048, 4096)
_CS0 = _cs_mat(4096, 64)
_AA0 = _aa_mat(64, 2048)
_CSM = _cs_mat(2048, 32)
_AAM = _aa_mat(32, 2048)
_CS4 = _cs_mat(2048, 64)
_AA4 = _aa_mat(64, 4096)

_gelu = jax.nn.gelu


def _mode_mix(xf, wr, wi, m):
    xr = xf[:, :m][:, None, :]
    xi = xf[:, m:][:, None, :]
    orr = jnp.sum(wr * xr - wi * xi, axis=0)
    oim = jnp.sum(wr * xi + wi * xr, axis=0)
    return jnp.concatenate([orr, oim], axis=1)


def _dot(a, b):
    return jnp.dot(a, b, preferred_element_type=_F32)


def _l0_body(x_ref, lw1, lb1, lw2, lb2, rd, cs, wr, wi, aa, wsm, bsm,
             w1, b1, w2, b2, o_ref):
    x = x_ref[0]
    l1 = _gelu(_dot(lw1[...], x) + lb1[...])
    h = _dot(lw2[...], l1) + lb2[...]
    hds = _dot(h.astype(_BF16), rd[...])
    xf = _dot(h, cs[...])
    y = _dot(_mode_mix(xf, wr[...], wi[...], 64), aa[...])
    sk = _dot(wsm[...], hds) + bsm[...]
    h0 = _gelu(y + sk[:64])
    t = _gelu(_dot(w1[...], h0) + b1[...])
    o_ref[0] = _gelu(_dot(w2[...], t) + b2[...] + sk[64:])


def _mid_body(h_ref, cs, wr, wi, aa, wsm, bsm, w1, b1, w2, b2, o_ref):
    h = h_ref[0]
    m = wr.shape[-1]
    co = wr.shape[1]
    xf = _dot(h, cs[...])
    y = _dot(_mode_mix(xf, wr[...], wi[...], m), aa[...])
    sk = _dot(wsm[...], h) + bsm[...]
    h0 = _gelu(y + sk[:co])
    t = _gelu(_dot(w1[...], h0) + b1[...])
    o_ref[0] = _gelu(_dot(w2[...], t) + b2[...] + sk[co:])


def _l4_body(h_ref, s0_ref, hw, hb, wsm, bsm, ru, cs, wr, wi, aa,
             w1, b1, w2, b2, pw1, pb1, pw2, pb2, o_ref):
    h = h_ref[0]
    s0 = s0_ref[0]
    hs = _dot(hw[...], s0) + hb[...]
    hcat = jnp.concatenate([h, hs], axis=0)
    sk2 = _dot(wsm[...], hcat) + bsm[...]
    sku = _dot(sk2.astype(_BF16), ru[...])
    xf = _dot(hcat, cs[...])
    y = _dot(_mode_mix(xf, wr[...], wi[...], 64), aa[...])
    h0 = y + sku[:64]
    t = _gelu(_dot(w1[...], h0) + b1[...])
    h4 = _dot(w2[...], t) + b2[...] + sku[64:]
    p1 = _gelu(_dot(pw1[...], h4) + pb1[...])
    o_ref[0] = _dot(pw2[...], p1) + pb2[...]


def _spec_b(shape):
    nd = len(shape)
    return pl.BlockSpec((1,) + tuple(shape[1:]),
                        lambda b, _n=nd: (b,) + (0,) * (_n - 1))


def _spec_c(shape):
    nd = len(shape)
    return pl.BlockSpec(tuple(shape), lambda b, _n=nd: (0,) * _n)


def _run(body, args, out_shape):
    in_specs = [(_spec_b if a.shape[:1] == (_B,) and a.ndim == 3 else _spec_c)(a.shape)
                for a in args]
    return pl.pallas_call(
        body,
        grid=(_B,),
        in_specs=in_specs,
        out_specs=_spec_b(out_shape),
        out_shape=jax.ShapeDtypeStruct(out_shape, _F32),
        compiler_params=pltpu.CompilerParams(
            dimension_semantics=("parallel",),
            vmem_limit_bytes=56 * 1024 * 1024,
        ),
    )(*args)


def _col(v):
    return v.reshape(-1, 1).astype(_F32)


def kernel(x, params):
    p = params
    lyr = p["layers"]

    def stacked(lp):
        wsm = jnp.concatenate([lp["skip_w"], lp["mskip_w"]], axis=0)
        bsm = jnp.concatenate([lp["skip_b"], lp["mskip_b"]]).reshape(-1, 1)
        return wsm, bsm

    rd = jnp.asarray(_R_DOWN, _BF16)
    ru = jnp.asarray(_R_UP, _BF16)

    wsm0, bsm0 = stacked(lyr[0])
    skip0 = _run(_l0_body, (
        x, p["lift_w1"], _col(p["lift_b1"]), p["lift_w2"], _col(p["lift_b2"]),
        rd, jnp.asarray(_CS0), lyr[0]["wr"], lyr[0]["wi"], jnp.asarray(_AA0),
        wsm0, bsm0, lyr[0]["mlp_w1"], _col(lyr[0]["mlp_b1"]),
        lyr[0]["mlp_w2"], _col(lyr[0]["mlp_b2"]),
    ), (_B, 64, 2048))

    h = skip0
    csm = jnp.asarray(_CSM)
    aam = jnp.asarray(_AAM)
    for i in (1, 2, 3):
        wsm, bsm = stacked(lyr[i])
        co = lyr[i]["skip_w"].shape[0]
        h = _run(_mid_body, (
            h, csm, lyr[i]["wr"], lyr[i]["wi"], aam, wsm, bsm,
            lyr[i]["mlp_w1"], _col(lyr[i]["mlp_b1"]),
            lyr[i]["mlp_w2"], _col(lyr[i]["mlp_b2"]),
        ), (_B, co, 2048))

    wsm4, bsm4 = stacked(lyr[4])
    out = _run(_l4_body, (
        h, skip0, p["horiz_w"], _col(p["horiz_b"]), wsm4, bsm4,
        ru, jnp.asarray(_CS4), lyr[4]["wr"], lyr[4]["wi"], jnp.asarray(_AA4),
        lyr[4]["mlp_w1"], _col(lyr[4]["mlp_b1"]),
        lyr[4]["mlp_w2"], _col(lyr[4]["mlp_b2"]),
        p["proj_w1"], _col(p["proj_b1"]), p["proj_w2"], _col(p["proj_b2"]),
    ), (_B, 1, 4096))
    return out
